# baseline (speedup 1.0000x reference)
import functools

import jax
import jax.numpy as jnp
from jax import lax
from jax.experimental import pallas as pl
from jax.experimental.pallas import tpu as pltpu
from jax.experimental.pallas import tpu_sc as plsc

NUM_CLASSES = 20000
BANK = 10
NQ = 5
NS = 2
D = 128
RPL = NQ * NS
B = 16
L = 50
T = 256
LPW = 2
AW = L // LPW
SLAB = LPW * RPL * B


def _sc_gather_add(table, vw, labT, widx):
    mesh = plsc.VectorSubcoreMesh(core_axis_name="c", subcore_axis_name="s")

    @functools.partial(
        pl.kernel,
        mesh=mesh,
        out_type=jax.ShapeDtypeStruct((L * RPL * B, D), jnp.float32),
        scratch_types=[
            pltpu.VMEM((LPW, B), jnp.int32),
            pltpu.VMEM((NQ, LPW * B), jnp.int32),
            pltpu.VMEM((1, 32), jnp.int32),
            pltpu.VMEM((LPW * B, NS, D), jnp.float32),
            pltpu.VMEM((LPW * B, NS, D), jnp.float32),
            pltpu.VMEM((32, D), jnp.float32),
            pltpu.VMEM((SLAB, D), jnp.float32),
            pltpu.SemaphoreType.DMA,
            pltpu.SemaphoreType.DMA,
            pltpu.SemaphoreType.DMA,
        ],
    )
    def k(table_hbm, vw_hbm, labT_hbm, widx_hbm, out_hbm,
          lab_v, idx_v, widx_v, rows_a, rows_b, wrows_v, acc_v,
          sem_a, sem_b, wsem):
        wid = lax.axis_index("s") * 2 + lax.axis_index("c")

        @pl.when(wid < AW)
        def _():
            pltpu.sync_copy(labT_hbm.at[wid], lab_v)
            pltpu.sync_copy(widx_hbm.at[wid], widx_v)
            wcopy = pltpu.async_copy(vw_hbm.at[widx_v.at[0]], wrows_v, wsem)
            for kk in range(NQ):
                for c in range(LPW):
                    idx_v[kk, pl.ds(c * B, B)] = lab_v[c] * BANK + kk
            bufs = (rows_a, rows_b)
            sems = (sem_a, sem_b)

            def gather(kk):
                return pltpu.async_copy(table_hbm.at[idx_v.at[kk]],
                                        bufs[kk % 2], sems[kk % 2])

            def densify_add(kk, i):
                rows = bufs[kk % 2]
                for c in range(LPW):
                    for s in range(NS):
                        lt = c * RPL + kk * NS + s
                        for j in range(D // 16):
                            sl = pl.ds(j * 16, 16)
                            acc_v[lt * B + i, sl] = (
                                wrows_v[lt, sl] + rows[c * B + i, s, sl])

            copies = [gather(0), gather(1)]
            wcopy.wait()
            for kk in range(NQ):
                copies[kk].wait()
                lax.fori_loop(
                    0, B, lambda i, _, kk=kk: (densify_add(kk, i), 0)[1], 0)
                if kk + 2 < NQ:
                    copies.append(gather(kk + 2))
            pltpu.sync_copy(acc_v, out_hbm.at[pl.ds(wid * SLAB, SLAB)])

    return k(table, vw, labT, widx)


def _tc_mask(locT):
    def body(loc_ref, m_ref):
        m = (loc_ref[...] != 0).astype(jnp.float32)
        m_ref[...] = jnp.broadcast_to(m[:, None, :, :],
                                      (L, RPL, B, T)).reshape(L * RPL, B, T)

    return pl.pallas_call(
        body,
        grid=(1,),
        in_specs=[pl.BlockSpec((L, B, T), lambda g: (0, 0, 0))],
        out_specs=pl.BlockSpec((L * RPL, B, T), lambda g: (0, 0, 0)),
        out_shape=jax.ShapeDtypeStruct((L * RPL, B, T), jnp.float32),
    )(locT)


def kernel(batched_label_list, batched_location_map, query_bank, vision_weight):
    table = query_bank.reshape(NUM_CLASSES * BANK, NS, D)

    labT = batched_label_list.astype(jnp.int32).T.reshape(AW, LPW, B)
    widx = (jnp.arange(AW, dtype=jnp.int32)[:, None] * (LPW * RPL)
            + jnp.arange(32, dtype=jnp.int32)[None, :] % (LPW * RPL))
    widx = widx.reshape(AW, 1, 32)

    q2d = _sc_gather_add(table, vision_weight, labT, widx)
    locT = batched_location_map.transpose(1, 0, 2)
    maskT = _tc_mask(locT)

    return (
        q2d.reshape(L * RPL, B, D).transpose(1, 0, 2),
        maskT.transpose(1, 0, 2),
        jnp.ones((B, L), dtype=jnp.int32),
    )
